# baseline (speedup 1.0000x reference)
import functools

import jax
import jax.numpy as jnp
from jax import lax
from jax.experimental import pallas as pl
from jax.experimental.pallas import tpu as pltpu
from jax.experimental.pallas import tpu_sc as plsc

SEQ_LEN = 8192
BATCH = 2
DIM = 1024
NUM_CORES = 2
NUM_SUBCORES = 16
NUM_WORKERS = NUM_CORES * NUM_SUBCORES
ROWS_PER_WORKER = SEQ_LEN // NUM_WORKERS
CHUNK = 8
NCHUNKS = ROWS_PER_WORKER // CHUNK
NSLOT = 3
LANES = 16


def _sc_body(emb_hbm, pe_hbm, out_hbm,
             eb0, eb1, eb2, pb0, pb1, pb2, ob0, ob1, ob2,
             sei0, sei1, sei2, spi0, spi1, spi2, so0, so1, so2):
    wid = lax.axis_index("s") * NUM_CORES + lax.axis_index("c")
    base = wid * ROWS_PER_WORKER
    ebufs, pbufs, obufs = (eb0, eb1, eb2), (pb0, pb1, pb2), (ob0, ob1, ob2)
    sei, spi, so = (sei0, sei1, sei2), (spi0, spi1, spi2), (so0, so1, so2)

    def start_in(g, s):
        r0 = base + g * CHUNK
        pltpu.async_copy(emb_hbm.at[pl.ds(r0, CHUNK)], ebufs[s], sei[s])
        pltpu.async_copy(pe_hbm.at[pl.ds(r0, CHUNK)], pbufs[s], spi[s])

    def wait_in(s):
        pltpu.make_async_copy(emb_hbm.at[pl.ds(0, CHUNK)], ebufs[s], sei[s]).wait()
        pltpu.make_async_copy(pe_hbm.at[pl.ds(0, CHUNK)], pbufs[s], spi[s]).wait()

    def start_out(g, s):
        r0 = base + g * CHUNK
        pltpu.async_copy(obufs[s], out_hbm.at[pl.ds(r0, CHUNK)], so[s])

    def wait_out(s):
        pltpu.make_async_copy(obufs[s], out_hbm.at[pl.ds(0, CHUNK)], so[s]).wait()

    def compute(s):
        eb, pb, ob = ebufs[s], pbufs[s], obufs[s]

        def row_step(r, c):
            for j in range(DIM // LANES):
                pv = pb[r, pl.ds(j * LANES, LANES)]
                ob[r, 0, pl.ds(j * LANES, LANES)] = (
                    eb[r, 0, pl.ds(j * LANES, LANES)] + pv)
                ob[r, 1, pl.ds(j * LANES, LANES)] = (
                    eb[r, 1, pl.ds(j * LANES, LANES)] + pv)
            return c

        lax.fori_loop(0, CHUNK, row_step, 0)

    for g in range(NSLOT):
        start_in(g, g)

    for s in range(NSLOT):
        wait_in(s)
        compute(s)
        start_out(s, s)
        start_in(NSLOT + s, s)

    def round_body(i, c):
        for s in range(NSLOT):
            g = NSLOT * i + s
            wait_out(s)
            wait_in(s)
            compute(s)
            start_out(g, s)
            start_in(g + NSLOT, s)
        return c

    n_full = (NCHUNKS - NSLOT) // NSLOT
    lax.fori_loop(1, n_full, round_body, 0)

    for g in range(NSLOT * n_full, NCHUNKS):
        s = g % NSLOT
        wait_out(s)
        wait_in(s)
        compute(s)
        start_out(g, s)
        if g + NSLOT < NCHUNKS:
            start_in(g + NSLOT, s)
    for s in range(NSLOT):
        wait_out(s)


@jax.jit
def kernel(emb, pe_table):
    seq_len, batch, dim = emb.shape
    sc_kernel = functools.partial(
        pl.kernel,
        out_type=jax.ShapeDtypeStruct((seq_len, batch, dim), emb.dtype),
        mesh=plsc.VectorSubcoreMesh(core_axis_name="c", subcore_axis_name="s"),
        scratch_types=(
            [pltpu.VMEM((CHUNK, BATCH, DIM), jnp.float32)] * 3 +
            [pltpu.VMEM((CHUNK, DIM), jnp.float32)] * 3 +
            [pltpu.VMEM((CHUNK, BATCH, DIM), jnp.float32)] * 3 +
            [pltpu.SemaphoreType.DMA] * 9
        ),
    )(_sc_body)
    return sc_kernel(emb, pe_table)
